# baseline (speedup 1.0000x reference)
import functools

import jax
import jax.numpy as jnp
from jax import lax
from jax.experimental import pallas as pl
from jax.experimental.pallas import tpu as pltpu
from jax.experimental.pallas import tpu_sc as plsc

_B = 27
_B1 = 16
_B2 = _B - _B1
_D = 128


def _gather_kernel(table_hbm, idx_hbm, out_hbm, idx_v, rows_v, sg1, sg2, so1, so2):
    pltpu.sync_copy(idx_hbm, idx_v)
    g1 = pltpu.async_copy(
        table_hbm.at[idx_v.at[pl.ds(0, _B1)]], rows_v.at[pl.ds(0, _B1)], sg1
    )
    g2 = pltpu.async_copy(
        table_hbm.at[idx_v.at[pl.ds(_B1, _B2)]], rows_v.at[pl.ds(_B1, _B2)], sg2
    )
    g1.wait()
    o1 = pltpu.async_copy(
        rows_v.at[pl.ds(0, _B1)], out_hbm.at[pl.ds(0, _B1)], so1
    )
    g2.wait()
    o2 = pltpu.async_copy(
        rows_v.at[pl.ds(_B1, _B2)], out_hbm.at[pl.ds(_B1, _B2)], so2
    )
    o1.wait()
    o2.wait()


def kernel(input, index):
    flat_idx = index.reshape(_B)
    mesh = plsc.VectorSubcoreMesh(
        core_axis_name="c", subcore_axis_name="s", num_cores=1, num_subcores=1
    )
    run = functools.partial(
        pl.kernel,
        mesh=mesh,
        out_type=jax.ShapeDtypeStruct((_B, _D), jnp.float32),
        scratch_types=[
            pltpu.VMEM((_B,), jnp.int32),
            pltpu.VMEM((_B, _D), jnp.float32),
            pltpu.SemaphoreType.DMA,
            pltpu.SemaphoreType.DMA,
            pltpu.SemaphoreType.DMA,
            pltpu.SemaphoreType.DMA,
        ],
        compiler_params=pltpu.CompilerParams(use_tc_tiling_on_sc=False),
    )(_gather_kernel)
    out = run(input, flat_idx)
    return out.reshape(index.shape + (_D,))
